# baseline (speedup 1.0000x reference)
import functools
import math

import jax
import jax.numpy as jnp
from jax import lax
from jax.experimental import pallas as pl
from jax.experimental.pallas import tpu as pltpu
from jax.experimental.pallas import tpu_sc as plsc

C_S = 256
C_Z = 128
C_HID = 256
N_NODES = 10000
K_NBR = 30
N_EDGES = N_NODES * K_NBR

E_PAD = 303104
E_HALF = E_PAD // 2
NR = 10240
TRASH_ROW = N_NODES + 64

BN = 256
BE = 512

_SC_CHUNK = 128
_GW = 32
_G_EW = E_PAD // _GW
_G_CH = _G_EW // _SC_CHUNK
_S_EW = E_PAD // 16
_S_CH = _S_EW // _SC_CHUNK
_WB = NR // 16
_WB_CH = _WB // _SC_CHUNK


def _ln(x, g, b, eps=1e-5):
    mu = jnp.mean(x, axis=-1, keepdims=True)
    var = jnp.mean((x - mu) ** 2, axis=-1, keepdims=True)
    return (x - mu) / jnp.sqrt(var + eps) * g + b


def _pack_bf16(x):
    a = jax.lax.convert_element_type(x[:, :128], jnp.bfloat16)
    b = jax.lax.convert_element_type(x[:, 128:], jnp.bfloat16)
    au = jax.lax.convert_element_type(
        jax.lax.bitcast_convert_type(a, jnp.uint16), jnp.uint32)
    bu = jax.lax.convert_element_type(
        jax.lax.bitcast_convert_type(b, jnp.uint16), jnp.uint32)
    return jax.lax.bitcast_convert_type((au << 16) | bu, jnp.int32)


def _unpack_hi(x):
    xu = jax.lax.bitcast_convert_type(x, jnp.uint32)
    return jax.lax.bitcast_convert_type(xu & jnp.uint32(0xFFFF0000),
                                        jnp.float32)


def _unpack_lo(x):
    xu = jax.lax.bitcast_convert_type(x, jnp.uint32)
    return jax.lax.bitcast_convert_type(xu << 16, jnp.float32)


def _full(shape):
    nd = len(shape)
    return pl.BlockSpec(shape, lambda i, _nd=nd: (0,) * _nd)


def _node_embed_body(t_ref, wt_ref, nf_ref, lat_ref, Wt_ref, Wnf_ref,
                     Wlat_ref, bn1_ref, Wn2_ref, bn2_ref, Wn3_ref, bn3_ref,
                     gn_ref, bn_ref, Wms_ref, Wmd_ref, bm_ref,
                     node_out, ps_out, pd_out):
    z = t_ref[0, 0] * wt_ref[...] * (2.0 * math.pi)
    temb = jnp.concatenate([jnp.sin(z), jnp.cos(z)], axis=-1)
    h = (jnp.dot(temb, Wt_ref[...], preferred_element_type=jnp.float32)
         + jnp.dot(nf_ref[...], Wnf_ref[...], preferred_element_type=jnp.float32)
         + jnp.dot(lat_ref[...], Wlat_ref[...], preferred_element_type=jnp.float32)
         + bn1_ref[...])
    h = jax.nn.relu(h)
    h = jax.nn.relu(jnp.dot(h, Wn2_ref[...], preferred_element_type=jnp.float32)
                    + bn2_ref[...])
    acc = jnp.zeros((BN, C_S), jnp.float32)
    for g in range(5):
        hg = jnp.dot(h, Wn3_ref[g], preferred_element_type=jnp.float32) + bn3_ref[g]
        ng = _ln(hg, gn_ref[...], bn_ref[...])
        node_out[g] = ng
        acc = acc + ng
    s = acc * 0.2
    ps_out[...] = _pack_bf16(
        jnp.dot(s, Wms_ref[...], preferred_element_type=jnp.float32))
    pd_out[...] = _pack_bf16(
        jnp.dot(s, Wmd_ref[...], preferred_element_type=jnp.float32)
        + bm_ref[...])


def _node_embed(t, w_time, nf, lat, Wt, Wnf, Wlat, bn1, Wn2, bn2, Wn3, bn3,
                g_node, b_node, Wms, Wmd, bm):
    grid = (N_NODES + BN - 1) // BN
    return pl.pallas_call(
        _node_embed_body,
        grid=(grid,),
        in_specs=[
            _full((1, 1)), _full((1, 32)),
            pl.BlockSpec((BN, 6), lambda i: (i, 0)),
            pl.BlockSpec((BN, 128), lambda i: (i, 0)),
            _full((64, 512)), _full((6, 512)), _full((128, 512)),
            _full((1, 512)), _full((512, 512)), _full((1, 512)),
            _full((5, 512, 256)), _full((5, 1, 256)),
            _full((1, 256)), _full((1, 256)),
            _full((256, 256)), _full((256, 256)), _full((1, 256)),
        ],
        out_specs=[
            pl.BlockSpec((5, BN, C_S), lambda i: (0, i, 0)),
            pl.BlockSpec((BN, 128), lambda i: (i, 0)),
            pl.BlockSpec((BN, 128), lambda i: (i, 0)),
        ],
        out_shape=[
            jax.ShapeDtypeStruct((5, N_NODES, C_S), jnp.float32),
            jax.ShapeDtypeStruct((N_NODES, 128), jnp.int32),
            jax.ShapeDtypeStruct((N_NODES, 128), jnp.int32),
        ],
    )(t, w_time, nf, lat, Wt, Wnf, Wlat, bn1, Wn2, bn2, Wn3, bn3,
      g_node, b_node, Wms, Wmd, bm)


def _edge_embed_body(x_ref, We1_ref, be1_ref, We2_ref, be2_ref, We3_ref,
                     be3_ref, ge_ref, be_ref, e_out):
    h = jax.nn.relu(jnp.dot(x_ref[...], We1_ref[...],
                            preferred_element_type=jnp.float32) + be1_ref[...])
    h = jax.nn.relu(jnp.dot(h, We2_ref[...],
                            preferred_element_type=jnp.float32) + be2_ref[...])
    e = jnp.dot(h, We3_ref[...], preferred_element_type=jnp.float32) + be3_ref[...]
    e_out[...] = _ln(e, ge_ref[...], be_ref[...])


def _edge_embed(x, We1, be1, We2, be2, We3, be3, ge, be, n):
    grid = n // BE
    return pl.pallas_call(
        _edge_embed_body,
        grid=(grid,),
        in_specs=[
            pl.BlockSpec((BE, 64), lambda i: (i, 0)),
            _full((64, 256)), _full((1, 256)),
            _full((256, 256)), _full((1, 256)),
            _full((256, 128)), _full((1, 128)),
            _full((1, 128)), _full((1, 128)),
        ],
        out_specs=pl.BlockSpec((BE, C_Z), lambda i: (i, 0)),
        out_shape=jax.ShapeDtypeStruct((n, C_Z), jnp.float32),
    )(x, We1, be1, We2, be2, We3, be3, ge, be)


def _edge_layer_body(e_ref, gs_ref, gd_ref, Wme_ref, WeL_ref, ge_ref, be_ref,
                     ma_out, mb_out, e_out):
    eproj = jnp.dot(e_ref[...], Wme_ref[...], preferred_element_type=jnp.float32)
    xs = gs_ref[...]
    xd = gd_ref[...]
    ma = jax.nn.relu(_unpack_hi(xs) + _unpack_hi(xd) + eproj[:, :128])
    mb = jax.nn.relu(_unpack_lo(xs) + _unpack_lo(xd) + eproj[:, 128:])
    WeL = WeL_ref[...]
    e2 = (e_ref[...]
          + jnp.dot(ma, WeL[:128], preferred_element_type=jnp.float32)
          + jnp.dot(mb, WeL[128:], preferred_element_type=jnp.float32))
    e_out[...] = _ln(e2, ge_ref[...], be_ref[...])
    ma_out[...] = ma
    mb_out[...] = mb


def _edge_layer(e, gs, gd, Wme, WeL, ge, be, n):
    grid = n // BE
    return pl.pallas_call(
        _edge_layer_body,
        grid=(grid,),
        in_specs=[
            pl.BlockSpec((BE, C_Z), lambda i: (i, 0)),
            pl.BlockSpec((BE, 128), lambda i: (i, 0)),
            pl.BlockSpec((BE, 128), lambda i: (i, 0)),
            _full((128, 256)), _full((256, 128)),
            _full((1, 128)), _full((1, 128)),
        ],
        out_specs=[
            pl.BlockSpec((BE, 128), lambda i: (i, 0)),
            pl.BlockSpec((BE, 128), lambda i: (i, 0)),
            pl.BlockSpec((BE, C_Z), lambda i: (i, 0)),
        ],
        out_shape=[
            jax.ShapeDtypeStruct((n, 128), jnp.float32),
            jax.ShapeDtypeStruct((n, 128), jnp.float32),
            jax.ShapeDtypeStruct((n, C_Z), jnp.float32),
        ],
    )(e, gs, gd, Wme, WeL, ge, be)


def _node_update_body(final, node_ref, aa1_ref, ab1_ref, aa2_ref, ab2_ref,
                      WuA_ref, WuB_ref,
                      gn_ref, bn_ref, w1_ref, w2_ref, w3_ref,
                      node_out, o1_out, o2_out):
    aa = aa1_ref[...] + aa2_ref[...]
    ab = ab1_ref[...] + ab2_ref[...]
    upd = (jnp.dot(aa, WuA_ref[...], preferred_element_type=jnp.float32)
           + jnp.dot(ab, WuB_ref[...], preferred_element_type=jnp.float32))
    acc = jnp.zeros((BN, C_S), jnp.float32)
    if final:
        logits = jnp.zeros((BN, 32), jnp.float32)
    for g in range(5):
        ng = _ln(node_ref[g] + upd, gn_ref[...], bn_ref[...])
        node_out[g] = ng
        if final:
            logits = logits + jnp.dot(ng, w1_ref[g],
                                      preferred_element_type=jnp.float32)
        else:
            acc = acc + ng
    if final:
        o1_out[...] = logits + w3_ref[...]
        o2_out[...] = jnp.zeros((BN, 128), jnp.int32)
    else:
        s = acc * 0.2
        o1_out[...] = _pack_bf16(
            jnp.dot(s, w1_ref[...], preferred_element_type=jnp.float32))
        o2_out[...] = _pack_bf16(
            jnp.dot(s, w2_ref[...], preferred_element_type=jnp.float32)
            + w3_ref[...])


def _node_update(node, aggA1, aggB1, aggA2, aggB2, WuA, WuB, gn, bn,
                 w1, w2, w3, final):
    grid = (N_NODES + BN - 1) // BN
    if final:
        w1_spec = _full((5, 256, 32))
        w3_spec = _full((1, 32))
        o1_shape = jax.ShapeDtypeStruct((N_NODES, 32), jnp.float32)
        o1_spec = pl.BlockSpec((BN, 32), lambda i: (i, 0))
    else:
        w1_spec = _full((256, 256))
        w3_spec = _full((1, 256))
        o1_shape = jax.ShapeDtypeStruct((N_NODES, 128), jnp.int32)
        o1_spec = pl.BlockSpec((BN, 128), lambda i: (i, 0))
    return pl.pallas_call(
        functools.partial(_node_update_body, final),
        grid=(grid,),
        in_specs=[
            pl.BlockSpec((5, BN, C_S), lambda i: (0, i, 0)),
            pl.BlockSpec((BN, 128), lambda i: (i, 0)),
            pl.BlockSpec((BN, 128), lambda i: (i, 0)),
            pl.BlockSpec((BN, 128), lambda i: (i, 0)),
            pl.BlockSpec((BN, 128), lambda i: (i, 0)),
            _full((128, 256)), _full((128, 256)),
            _full((1, 256)), _full((1, 256)),
            w1_spec, _full((256, 256)), w3_spec,
        ],
        out_specs=[
            pl.BlockSpec((5, BN, C_S), lambda i: (0, i, 0)),
            o1_spec,
            pl.BlockSpec((BN, 128), lambda i: (i, 0)),
        ],
        out_shape=[
            jax.ShapeDtypeStruct((5, N_NODES, C_S), jnp.float32),
            o1_shape,
            jax.ShapeDtypeStruct((N_NODES, 128), jnp.int32),
        ],
    )(node, aggA1, aggB1, aggA2, aggB2, WuA, WuB, gn, bn, w1, w2, w3)


_GC = 128


def _sc_gather(ps, pd, srcp, dstp, n):
    g_ew = n // _GW
    g_ch = g_ew // _GC
    mesh = plsc.VectorSubcoreMesh(core_axis_name="c", subcore_axis_name="s")

    @functools.partial(
        pl.kernel,
        out_type=(jax.ShapeDtypeStruct((n, 128), jnp.int32),
                  jax.ShapeDtypeStruct((n, 128), jnp.int32)),
        mesh=mesh,
        scratch_types=[
            pltpu.VMEM((2, _GC), jnp.int32),
            pltpu.VMEM((2, _GC), jnp.int32),
            pltpu.VMEM((2, _GC, 128), jnp.int32),
            pltpu.VMEM((2, _GC, 128), jnp.int32),
            pltpu.SemaphoreType.DMA,
            pltpu.SemaphoreType.DMA,
            pltpu.SemaphoreType.DMA,
            pltpu.SemaphoreType.DMA,
            pltpu.SemaphoreType.DMA,
            pltpu.SemaphoreType.DMA,
        ],
    )
    def k(ps_hbm, pd_hbm, src_hbm, dst_hbm, gs_hbm, gd_hbm,
          idx_s, idx_d, buf_s, buf_d,
          sem_is, sem_id, sem_gs, sem_gd, sem_ws, sem_wd):
        wid = lax.axis_index("s") * 2 + lax.axis_index("c")
        base = wid * g_ew

        def start_idx(c, slot):
            off = base + c * _GC
            pltpu.async_copy(src_hbm.at[pl.ds(off, _GC)], idx_s.at[slot],
                             sem_is)
            pltpu.async_copy(dst_hbm.at[pl.ds(off, _GC)], idx_d.at[slot],
                             sem_id)

        def wait_idx(slot):
            pltpu.make_async_copy(src_hbm.at[pl.ds(base, _GC)],
                                  idx_s.at[slot], sem_is).wait()
            pltpu.make_async_copy(dst_hbm.at[pl.ds(base, _GC)],
                                  idx_d.at[slot], sem_id).wait()

        def start_gather(slot):
            pltpu.async_copy(ps_hbm.at[idx_s.at[slot]], buf_s.at[slot],
                             sem_gs)
            pltpu.async_copy(pd_hbm.at[idx_d.at[slot]], buf_d.at[slot],
                             sem_gd)

        def wait_gather(slot):
            pltpu.make_async_copy(ps_hbm.at[idx_s.at[slot]], buf_s.at[slot],
                                  sem_gs).wait()
            pltpu.make_async_copy(pd_hbm.at[idx_d.at[slot]], buf_d.at[slot],
                                  sem_gd).wait()

        def start_write(c, slot):
            off = base + c * _GC
            pltpu.async_copy(buf_s.at[slot], gs_hbm.at[pl.ds(off, _GC)],
                             sem_ws)
            pltpu.async_copy(buf_d.at[slot], gd_hbm.at[pl.ds(off, _GC)],
                             sem_wd)

        def wait_write(slot):
            pltpu.make_async_copy(buf_s.at[slot],
                                  gs_hbm.at[pl.ds(base, _GC)], sem_ws).wait()
            pltpu.make_async_copy(buf_d.at[slot],
                                  gd_hbm.at[pl.ds(base, _GC)], sem_wd).wait()

        off0 = base
        pltpu.sync_copy(src_hbm.at[pl.ds(off0, _GC)], idx_s.at[0])
        pltpu.sync_copy(dst_hbm.at[pl.ds(off0, _GC)], idx_d.at[0])
        start_gather(0)
        start_idx(1, 1)

        def body(c, carry):
            p = lax.rem(c, 2)
            q = 1 - p

            @pl.when(c + 1 < g_ch)
            def _():
                wait_idx(q)
                @pl.when(c >= 1)
                def _():
                    wait_write(q)
                start_gather(q)

            wait_gather(p)
            start_write(c, p)

            @pl.when(c + 2 < g_ch)
            def _():
                start_idx(c + 2, p)

            return carry

        lax.fori_loop(0, g_ch, body, 0)
        wait_write((g_ch - 1) % 2)
        wait_write(g_ch % 2)

    return k(ps, pd, srcp, dstp)


def _sc_scatter(ma, mb, dstp, n):
    s_ew = n // 16
    s_ch = s_ew // _SC_CHUNK
    assert s_ch % 2 == 0
    mesh = plsc.VectorSubcoreMesh(core_axis_name="c", subcore_axis_name="s")

    @functools.partial(
        pl.kernel,
        out_type=(jax.ShapeDtypeStruct((NR, 128), jnp.float32),
                  jax.ShapeDtypeStruct((NR, 128), jnp.float32)),
        mesh=mesh,
        scratch_types=[
            pltpu.VMEM((_SC_CHUNK,), jnp.int32),
            pltpu.VMEM((_SC_CHUNK,), jnp.int32),
            pltpu.VMEM((2, _SC_CHUNK, 128), jnp.float32),
            pltpu.VMEM_SHARED((NR, 128), jnp.float32),
            pltpu.SemaphoreType.DMA,
            pltpu.SemaphoreType.DMA,
            pltpu.SemaphoreType.DMA,
        ],
    )
    def k(ma_hbm, mb_hbm, dst_hbm,
          aa_hbm, ab_hbm, idx_a, idx_b, m_v, agg_sh,
          sem_i, sem_m, sem_sc):
        cid = lax.axis_index("c")
        sid = lax.axis_index("s")

        zeros16 = jnp.zeros((16,), jnp.float32)

        def zbody(i, carry):
            m_v[0, i // 8, pl.ds((i % 8) * 16, 16)] = zeros16
            return carry

        lax.fori_loop(0, _SC_CHUNK * 8, zbody, 0)

        def zcopy(kk, carry):
            pltpu.sync_copy(
                m_v.at[0],
                agg_sh.at[pl.ds(sid * _WB + kk * _SC_CHUNK, _SC_CHUNK)])
            return carry

        lax.fori_loop(0, _WB_CH, zcopy, 0)
        plsc.subcore_barrier()

        sbase = sid * s_ew

        def scatter_from(m_hbm):
            def start_loads(c, idx_ref, slot):
                off = sbase + c * _SC_CHUNK
                pltpu.async_copy(dst_hbm.at[pl.ds(off, _SC_CHUNK)],
                                 idx_ref, sem_i)
                pltpu.async_copy(m_hbm.at[pl.ds(off, _SC_CHUNK)],
                                 m_v.at[slot], sem_m)

            def wait_loads(idx_ref, slot):
                pltpu.make_async_copy(dst_hbm.at[pl.ds(sbase, _SC_CHUNK)],
                                      idx_ref, sem_i).wait()
                pltpu.make_async_copy(m_hbm.at[pl.ds(0, _SC_CHUNK)],
                                      m_v.at[slot], sem_m).wait()

            def fire_sc(idx_ref, slot):
                pltpu.async_copy(m_v.at[slot], agg_sh.at[idx_ref], sem_sc,
                                 add=True)

            def wait_sc():
                pltpu.make_async_copy(m_v.at[0], agg_sh.at[idx_a],
                                      sem_sc).wait()

            start_loads(0, idx_a, 0)

            def body(tt, carry):
                c = tt * 2

                wait_loads(idx_a, 0)
                fire_sc(idx_a, 0)

                @pl.when(tt >= 1)
                def _():
                    wait_sc()

                @pl.when(c + 1 < s_ch)
                def _():
                    start_loads(c + 1, idx_b, 1)
                    wait_loads(idx_b, 1)
                    fire_sc(idx_b, 1)
                    wait_sc()

                    @pl.when(c + 2 < s_ch)
                    def _():
                        start_loads(c + 2, idx_a, 0)

                return carry

            lax.fori_loop(0, s_ch // 2, body, 0)
            wait_sc()

        @pl.when(cid == 0)
        def _():
            scatter_from(ma_hbm)

        @pl.when(cid == 1)
        def _():
            scatter_from(mb_hbm)

        plsc.subcore_barrier()

        def writeback(out_hbm):
            def body(kk, carry):
                r = sid * _WB + kk * _SC_CHUNK
                pltpu.sync_copy(agg_sh.at[pl.ds(r, _SC_CHUNK)], m_v.at[0])
                pltpu.sync_copy(m_v.at[0], out_hbm.at[pl.ds(r, _SC_CHUNK)])
                return carry
            lax.fori_loop(0, _WB_CH, body, 0)

        @pl.when(cid == 0)
        def _():
            writeback(aa_hbm)

        @pl.when(cid == 1)
        def _():
            writeback(ab_hbm)

    return k(ma, mb, dstp)


def kernel(node_features, latent_sidechain, t, edge_raw, params, edge_index):
    p = params
    dst = edge_index[0]
    src = edge_index[1]

    pad_e = E_PAD - N_EDGES
    srcp = jnp.concatenate([src, jnp.zeros((pad_e,), jnp.int32)])
    dstp = jnp.concatenate([dst, jnp.full((pad_e,), TRASH_ROW, jnp.int32)])
    edge_raw_p = jnp.concatenate(
        [edge_raw, jnp.zeros((pad_e, 64), jnp.float32)], axis=0)

    Wn1 = p['Wn1']
    Wt, Wnf, Wlat = Wn1[:64], Wn1[64:70], Wn1[70:]
    Wn3 = p['Wn3'].reshape(512, 5, 256).transpose(1, 0, 2)
    bn3 = p['bn3'].reshape(5, 1, 256)
    W_head = p['W_head'].reshape(5, 256, 20)
    W_head = jnp.pad(W_head, ((0, 0), (0, 0), (0, 12)))
    b_head = jnp.pad(p['b_head'], (0, 12)).reshape(1, 32)

    r2 = lambda v: v.reshape(1, -1)
    lws = []
    for lp in p['layers']:
        Wm = lp['Wm']
        lws.append(dict(
            Wms=Wm[:256], Wmd=Wm[256:512], Wme=Wm[512:],
            bm=r2(lp['bm']), WuA=lp['Wu'][:128], WuB=lp['Wu'][128:],
            WeL=lp['We'], gn=r2(lp['gn']), bn=r2(lp['bn']),
            ge=r2(lp['ge']), be=r2(lp['be']),
        ))

    node, ps, pd = _node_embed(
        t.reshape(1, 1), p['w_time'].reshape(1, 32),
        node_features, latent_sidechain,
        Wt, Wnf, Wlat, r2(p['bn1']), p['Wn2'], r2(p['bn2']), Wn3, bn3,
        r2(p['g_node']), r2(p['b_node']),
        lws[0]['Wms'], lws[0]['Wmd'], lws[0]['bm'])

    srcp_h = (srcp[:E_HALF], srcp[E_HALF:])
    dstp_h = (dstp[:E_HALF], dstp[E_HALF:])

    g_h = [_sc_gather(ps, pd, srcp_h[0], dstp_h[0], E_HALF),
           _sc_gather(ps, pd, srcp_h[1], dstp_h[1], E_HALF)]

    e_h = [
        _edge_embed(edge_raw_p[h * E_HALF:(h + 1) * E_HALF],
                    p['We1'], r2(p['be1']), p['We2'],
                    r2(p['be2']), p['We3'], r2(p['be3']),
                    r2(p['g_edge']), r2(p['b_edge']), E_HALF)
        for h in range(2)
    ]

    for l in range(4):
        lw = lws[l]
        agg_h = [None, None]
        for h in range(2):
            ma, mb, e_new = _edge_layer(e_h[h], g_h[h][0], g_h[h][1],
                                        lw['Wme'], lw['WeL'],
                                        lw['ge'], lw['be'], E_HALF)
            e_h[h] = e_new
            agg_h[h] = _sc_scatter(ma, mb, dstp_h[h], E_HALF)
        if l < 3:
            nxt = lws[l + 1]
            node, ps, pd = _node_update(
                node, agg_h[0][0][:N_NODES], agg_h[0][1][:N_NODES],
                agg_h[1][0][:N_NODES], agg_h[1][1][:N_NODES],
                lw['WuA'], lw['WuB'], lw['gn'], lw['bn'],
                nxt['Wms'], nxt['Wmd'], nxt['bm'], final=False)
            g_h = [_sc_gather(ps, pd, srcp_h[0], dstp_h[0], E_HALF),
                   _sc_gather(ps, pd, srcp_h[1], dstp_h[1], E_HALF)]
        else:
            node, logits, _ = _node_update(
                node, agg_h[0][0][:N_NODES], agg_h[0][1][:N_NODES],
                agg_h[1][0][:N_NODES], agg_h[1][1][:N_NODES],
                lw['WuA'], lw['WuB'], lw['gn'], lw['bn'],
                W_head, lws[0]['Wms'], b_head, final=True)

    return logits[:, :20]


# speedup vs baseline: 1.0157x; 1.0157x over previous
import functools
import math

import jax
import jax.numpy as jnp
from jax import lax
from jax.experimental import pallas as pl
from jax.experimental.pallas import tpu as pltpu
from jax.experimental.pallas import tpu_sc as plsc

C_S = 256
C_Z = 128
C_HID = 256
N_NODES = 10000
K_NBR = 30
N_EDGES = N_NODES * K_NBR

E_PAD = 303104
E_HALF = E_PAD // 2
NR = 10240
TRASH_ROW = N_NODES + 64

BN = 256
BE = 512

_SC_CHUNK = 128
_GW = 32
_G_EW = E_PAD // _GW
_G_CH = _G_EW // _SC_CHUNK
_S_EW = E_PAD // 16
_S_CH = _S_EW // _SC_CHUNK
_WB = NR // 16
_WB_CH = _WB // _SC_CHUNK


def _ln(x, g, b, eps=1e-5):
    mu = jnp.mean(x, axis=-1, keepdims=True)
    var = jnp.mean((x - mu) ** 2, axis=-1, keepdims=True)
    return (x - mu) / jnp.sqrt(var + eps) * g + b


def _pack_bf16(x):
    a = jax.lax.convert_element_type(x[:, :128], jnp.bfloat16)
    b = jax.lax.convert_element_type(x[:, 128:], jnp.bfloat16)
    au = jax.lax.convert_element_type(
        jax.lax.bitcast_convert_type(a, jnp.uint16), jnp.uint32)
    bu = jax.lax.convert_element_type(
        jax.lax.bitcast_convert_type(b, jnp.uint16), jnp.uint32)
    return jax.lax.bitcast_convert_type((au << 16) | bu, jnp.int32)


def _unpack_hi(x):
    xu = jax.lax.bitcast_convert_type(x, jnp.uint32)
    return jax.lax.bitcast_convert_type(xu & jnp.uint32(0xFFFF0000),
                                        jnp.float32)


def _unpack_lo(x):
    xu = jax.lax.bitcast_convert_type(x, jnp.uint32)
    return jax.lax.bitcast_convert_type(xu << 16, jnp.float32)


def _full(shape):
    nd = len(shape)
    return pl.BlockSpec(shape, lambda i, _nd=nd: (0,) * _nd)


def _node_embed_body(t_ref, wt_ref, nf_ref, lat_ref, Wt_ref, Wnf_ref,
                     Wlat_ref, bn1_ref, Wn2_ref, bn2_ref, Wn3_ref, bn3_ref,
                     gn_ref, bn_ref, Wms_ref, Wmd_ref, bm_ref,
                     node_out, ps_out, pd_out):
    z = t_ref[0, 0] * wt_ref[...] * (2.0 * math.pi)
    temb = jnp.concatenate([jnp.sin(z), jnp.cos(z)], axis=-1)
    h = (jnp.dot(temb, Wt_ref[...], preferred_element_type=jnp.float32)
         + jnp.dot(nf_ref[...], Wnf_ref[...], preferred_element_type=jnp.float32)
         + jnp.dot(lat_ref[...], Wlat_ref[...], preferred_element_type=jnp.float32)
         + bn1_ref[...])
    h = jax.nn.relu(h)
    h = jax.nn.relu(jnp.dot(h, Wn2_ref[...], preferred_element_type=jnp.float32)
                    + bn2_ref[...])
    acc = jnp.zeros((BN, C_S), jnp.float32)
    for g in range(5):
        hg = jnp.dot(h, Wn3_ref[g], preferred_element_type=jnp.float32) + bn3_ref[g]
        ng = _ln(hg, gn_ref[...], bn_ref[...])
        node_out[g] = ng
        acc = acc + ng
    s = acc * 0.2
    ps_out[...] = _pack_bf16(
        jnp.dot(s, Wms_ref[...], preferred_element_type=jnp.float32))
    pd_out[...] = _pack_bf16(
        jnp.dot(s, Wmd_ref[...], preferred_element_type=jnp.float32)
        + bm_ref[...])


def _node_embed(t, w_time, nf, lat, Wt, Wnf, Wlat, bn1, Wn2, bn2, Wn3, bn3,
                g_node, b_node, Wms, Wmd, bm):
    grid = (N_NODES + BN - 1) // BN
    return pl.pallas_call(
        _node_embed_body,
        grid=(grid,),
        in_specs=[
            _full((1, 1)), _full((1, 32)),
            pl.BlockSpec((BN, 6), lambda i: (i, 0)),
            pl.BlockSpec((BN, 128), lambda i: (i, 0)),
            _full((64, 512)), _full((6, 512)), _full((128, 512)),
            _full((1, 512)), _full((512, 512)), _full((1, 512)),
            _full((5, 512, 256)), _full((5, 1, 256)),
            _full((1, 256)), _full((1, 256)),
            _full((256, 256)), _full((256, 256)), _full((1, 256)),
        ],
        out_specs=[
            pl.BlockSpec((5, BN, C_S), lambda i: (0, i, 0)),
            pl.BlockSpec((BN, 128), lambda i: (i, 0)),
            pl.BlockSpec((BN, 128), lambda i: (i, 0)),
        ],
        out_shape=[
            jax.ShapeDtypeStruct((5, N_NODES, C_S), jnp.float32),
            jax.ShapeDtypeStruct((N_NODES, 128), jnp.int32),
            jax.ShapeDtypeStruct((N_NODES, 128), jnp.int32),
        ],
    )(t, w_time, nf, lat, Wt, Wnf, Wlat, bn1, Wn2, bn2, Wn3, bn3,
      g_node, b_node, Wms, Wmd, bm)


def _edge_embed_body(x_ref, We1_ref, be1_ref, We2_ref, be2_ref, We3_ref,
                     be3_ref, ge_ref, be_ref, e_out):
    h = jax.nn.relu(jnp.dot(x_ref[...], We1_ref[...],
                            preferred_element_type=jnp.float32) + be1_ref[...])
    h = jax.nn.relu(jnp.dot(h, We2_ref[...],
                            preferred_element_type=jnp.float32) + be2_ref[...])
    e = jnp.dot(h, We3_ref[...], preferred_element_type=jnp.float32) + be3_ref[...]
    e_out[...] = _ln(e, ge_ref[...], be_ref[...])


def _edge_embed(x, We1, be1, We2, be2, We3, be3, ge, be, n):
    grid = n // BE
    return pl.pallas_call(
        _edge_embed_body,
        grid=(grid,),
        in_specs=[
            pl.BlockSpec((BE, 64), lambda i: (i, 0)),
            _full((64, 256)), _full((1, 256)),
            _full((256, 256)), _full((1, 256)),
            _full((256, 128)), _full((1, 128)),
            _full((1, 128)), _full((1, 128)),
        ],
        out_specs=pl.BlockSpec((BE, C_Z), lambda i: (i, 0)),
        out_shape=jax.ShapeDtypeStruct((n, C_Z), jnp.float32),
    )(x, We1, be1, We2, be2, We3, be3, ge, be)


def _edge_layer_body(e_ref, gs_ref, gd_ref, Wme_ref, WeL_ref, ge_ref, be_ref,
                     ma_out, mb_out, e_out):
    eproj = jnp.dot(e_ref[...], Wme_ref[...], preferred_element_type=jnp.float32)
    xs = gs_ref[...]
    xd = gd_ref[...]
    ma = jax.nn.relu(_unpack_hi(xs) + _unpack_hi(xd) + eproj[:, :128])
    mb = jax.nn.relu(_unpack_lo(xs) + _unpack_lo(xd) + eproj[:, 128:])
    WeL = WeL_ref[...]
    e2 = (e_ref[...]
          + jnp.dot(ma, WeL[:128], preferred_element_type=jnp.float32)
          + jnp.dot(mb, WeL[128:], preferred_element_type=jnp.float32))
    e_out[...] = _ln(e2, ge_ref[...], be_ref[...])
    ma_out[...] = ma
    mb_out[...] = mb


def _edge_layer(e, gs, gd, Wme, WeL, ge, be, n):
    grid = n // BE
    return pl.pallas_call(
        _edge_layer_body,
        grid=(grid,),
        in_specs=[
            pl.BlockSpec((BE, C_Z), lambda i: (i, 0)),
            pl.BlockSpec((BE, 128), lambda i: (i, 0)),
            pl.BlockSpec((BE, 128), lambda i: (i, 0)),
            _full((128, 256)), _full((256, 128)),
            _full((1, 128)), _full((1, 128)),
        ],
        out_specs=[
            pl.BlockSpec((BE, 128), lambda i: (i, 0)),
            pl.BlockSpec((BE, 128), lambda i: (i, 0)),
            pl.BlockSpec((BE, C_Z), lambda i: (i, 0)),
        ],
        out_shape=[
            jax.ShapeDtypeStruct((n, 128), jnp.float32),
            jax.ShapeDtypeStruct((n, 128), jnp.float32),
            jax.ShapeDtypeStruct((n, C_Z), jnp.float32),
        ],
    )(e, gs, gd, Wme, WeL, ge, be)


def _node_update_body(final, node_ref, aa1_ref, ab1_ref, aa2_ref, ab2_ref,
                      WuA_ref, WuB_ref,
                      gn_ref, bn_ref, w1_ref, w2_ref, w3_ref,
                      node_out, o1_out, o2_out):
    aa = aa1_ref[...] + aa2_ref[...]
    ab = ab1_ref[...] + ab2_ref[...]
    upd = (jnp.dot(aa, WuA_ref[...], preferred_element_type=jnp.float32)
           + jnp.dot(ab, WuB_ref[...], preferred_element_type=jnp.float32))
    acc = jnp.zeros((BN, C_S), jnp.float32)
    if final:
        logits = jnp.zeros((BN, 32), jnp.float32)
    for g in range(5):
        ng = _ln(node_ref[g] + upd, gn_ref[...], bn_ref[...])
        node_out[g] = ng
        if final:
            logits = logits + jnp.dot(ng, w1_ref[g],
                                      preferred_element_type=jnp.float32)
        else:
            acc = acc + ng
    if final:
        o1_out[...] = logits + w3_ref[...]
        o2_out[...] = jnp.zeros((BN, 128), jnp.int32)
    else:
        s = acc * 0.2
        o1_out[...] = _pack_bf16(
            jnp.dot(s, w1_ref[...], preferred_element_type=jnp.float32))
        o2_out[...] = _pack_bf16(
            jnp.dot(s, w2_ref[...], preferred_element_type=jnp.float32)
            + w3_ref[...])


def _node_update(node, aggA1, aggB1, aggA2, aggB2, WuA, WuB, gn, bn,
                 w1, w2, w3, final):
    grid = (N_NODES + BN - 1) // BN
    if final:
        w1_spec = _full((5, 256, 32))
        w3_spec = _full((1, 32))
        o1_shape = jax.ShapeDtypeStruct((N_NODES, 32), jnp.float32)
        o1_spec = pl.BlockSpec((BN, 32), lambda i: (i, 0))
    else:
        w1_spec = _full((256, 256))
        w3_spec = _full((1, 256))
        o1_shape = jax.ShapeDtypeStruct((N_NODES, 128), jnp.int32)
        o1_spec = pl.BlockSpec((BN, 128), lambda i: (i, 0))
    return pl.pallas_call(
        functools.partial(_node_update_body, final),
        grid=(grid,),
        in_specs=[
            pl.BlockSpec((5, BN, C_S), lambda i: (0, i, 0)),
            pl.BlockSpec((BN, 128), lambda i: (i, 0)),
            pl.BlockSpec((BN, 128), lambda i: (i, 0)),
            pl.BlockSpec((BN, 128), lambda i: (i, 0)),
            pl.BlockSpec((BN, 128), lambda i: (i, 0)),
            _full((128, 256)), _full((128, 256)),
            _full((1, 256)), _full((1, 256)),
            w1_spec, _full((256, 256)), w3_spec,
        ],
        out_specs=[
            pl.BlockSpec((5, BN, C_S), lambda i: (0, i, 0)),
            o1_spec,
            pl.BlockSpec((BN, 128), lambda i: (i, 0)),
        ],
        out_shape=[
            jax.ShapeDtypeStruct((5, N_NODES, C_S), jnp.float32),
            o1_shape,
            jax.ShapeDtypeStruct((N_NODES, 128), jnp.int32),
        ],
    )(node, aggA1, aggB1, aggA2, aggB2, WuA, WuB, gn, bn, w1, w2, w3)


_GC = 128


def _sc_gather(ps, pd, srcp, dstp, n):
    g_ew = n // _GW
    g_ch = g_ew // _GC
    mesh = plsc.VectorSubcoreMesh(core_axis_name="c", subcore_axis_name="s")

    @functools.partial(
        pl.kernel,
        out_type=(jax.ShapeDtypeStruct((n, 128), jnp.int32),
                  jax.ShapeDtypeStruct((n, 128), jnp.int32)),
        mesh=mesh,
        scratch_types=[
            pltpu.VMEM((2, _GC), jnp.int32),
            pltpu.VMEM((2, _GC), jnp.int32),
            pltpu.VMEM((2, _GC, 128), jnp.int32),
            pltpu.VMEM((2, _GC, 128), jnp.int32),
            pltpu.SemaphoreType.DMA,
            pltpu.SemaphoreType.DMA,
            pltpu.SemaphoreType.DMA,
            pltpu.SemaphoreType.DMA,
            pltpu.SemaphoreType.DMA,
            pltpu.SemaphoreType.DMA,
        ],
    )
    def k(ps_hbm, pd_hbm, src_hbm, dst_hbm, gs_hbm, gd_hbm,
          idx_s, idx_d, buf_s, buf_d,
          sem_is, sem_id, sem_gs, sem_gd, sem_ws, sem_wd):
        wid = lax.axis_index("s") * 2 + lax.axis_index("c")
        base = wid * g_ew

        def start_idx(c, slot):
            off = base + c * _GC
            pltpu.async_copy(src_hbm.at[pl.ds(off, _GC)], idx_s.at[slot],
                             sem_is)
            pltpu.async_copy(dst_hbm.at[pl.ds(off, _GC)], idx_d.at[slot],
                             sem_id)

        def wait_idx(slot):
            pltpu.make_async_copy(src_hbm.at[pl.ds(base, _GC)],
                                  idx_s.at[slot], sem_is).wait()
            pltpu.make_async_copy(dst_hbm.at[pl.ds(base, _GC)],
                                  idx_d.at[slot], sem_id).wait()

        def start_gather(slot):
            pltpu.async_copy(ps_hbm.at[idx_s.at[slot]], buf_s.at[slot],
                             sem_gs)
            pltpu.async_copy(pd_hbm.at[idx_d.at[slot]], buf_d.at[slot],
                             sem_gd)

        def wait_gather(slot):
            pltpu.make_async_copy(ps_hbm.at[idx_s.at[slot]], buf_s.at[slot],
                                  sem_gs).wait()
            pltpu.make_async_copy(pd_hbm.at[idx_d.at[slot]], buf_d.at[slot],
                                  sem_gd).wait()

        def start_write(c, slot):
            off = base + c * _GC
            pltpu.async_copy(buf_s.at[slot], gs_hbm.at[pl.ds(off, _GC)],
                             sem_ws)
            pltpu.async_copy(buf_d.at[slot], gd_hbm.at[pl.ds(off, _GC)],
                             sem_wd)

        def wait_write(slot):
            pltpu.make_async_copy(buf_s.at[slot],
                                  gs_hbm.at[pl.ds(base, _GC)], sem_ws).wait()
            pltpu.make_async_copy(buf_d.at[slot],
                                  gd_hbm.at[pl.ds(base, _GC)], sem_wd).wait()

        off0 = base
        pltpu.sync_copy(src_hbm.at[pl.ds(off0, _GC)], idx_s.at[0])
        pltpu.sync_copy(dst_hbm.at[pl.ds(off0, _GC)], idx_d.at[0])
        start_gather(0)
        start_idx(1, 1)

        def body(c, carry):
            p = lax.rem(c, 2)
            q = 1 - p

            @pl.when(c + 1 < g_ch)
            def _():
                wait_idx(q)
                @pl.when(c >= 1)
                def _():
                    wait_write(q)
                start_gather(q)

            wait_gather(p)
            start_write(c, p)

            @pl.when(c + 2 < g_ch)
            def _():
                start_idx(c + 2, p)

            return carry

        lax.fori_loop(0, g_ch, body, 0)
        wait_write((g_ch - 1) % 2)
        wait_write(g_ch % 2)

    return k(ps, pd, srcp, dstp)


def _sc_scatter(ma, mb, dstp, n):
    s_ew = n // 16
    s_ch = s_ew // _SC_CHUNK
    assert s_ch % 2 == 0
    mesh = plsc.VectorSubcoreMesh(core_axis_name="c", subcore_axis_name="s")

    @functools.partial(
        pl.kernel,
        out_type=(jax.ShapeDtypeStruct((NR, 128), jnp.float32),
                  jax.ShapeDtypeStruct((NR, 128), jnp.float32)),
        mesh=mesh,
        scratch_types=[
            pltpu.VMEM((_SC_CHUNK,), jnp.int32),
            pltpu.VMEM((_SC_CHUNK,), jnp.int32),
            pltpu.VMEM((2, _SC_CHUNK, 128), jnp.float32),
            pltpu.VMEM_SHARED((NR, 128), jnp.float32),
            pltpu.SemaphoreType.DMA,
            pltpu.SemaphoreType.DMA,
        ],
    )
    def k(ma_hbm, mb_hbm, dst_hbm,
          aa_hbm, ab_hbm, idx_a, idx_b, m_v, agg_sh,
          sem_i, sem_m):
        cid = lax.axis_index("c")
        sid = lax.axis_index("s")

        zeros16 = jnp.zeros((16,), jnp.float32)

        def zbody(i, carry):
            m_v[0, i // 8, pl.ds((i % 8) * 16, 16)] = zeros16
            return carry

        lax.fori_loop(0, _SC_CHUNK * 8, zbody, 0)

        def zcopy(kk, carry):
            pltpu.sync_copy(
                m_v.at[0],
                agg_sh.at[pl.ds(sid * _WB + kk * _SC_CHUNK, _SC_CHUNK)])
            return carry

        lax.fori_loop(0, _WB_CH, zcopy, 0)
        plsc.subcore_barrier()

        sbase = sid * s_ew

        def scatter_from(m_hbm):
            def start_loads(c, idx_ref, slot):
                off = sbase + c * _SC_CHUNK
                pltpu.async_copy(dst_hbm.at[pl.ds(off, _SC_CHUNK)],
                                 idx_ref, sem_i)
                pltpu.async_copy(m_hbm.at[pl.ds(off, _SC_CHUNK)],
                                 m_v.at[slot], sem_m)

            def wait_loads(idx_ref, slot):
                pltpu.make_async_copy(dst_hbm.at[pl.ds(sbase, _SC_CHUNK)],
                                      idx_ref, sem_i).wait()
                pltpu.make_async_copy(m_hbm.at[pl.ds(0, _SC_CHUNK)],
                                      m_v.at[slot], sem_m).wait()

            start_loads(0, idx_a, 0)

            def body(tt, carry):
                c = tt * 2

                start_loads(c + 1, idx_b, 1)
                wait_loads(idx_a, 0)
                pltpu.sync_copy(m_v.at[0], agg_sh.at[idx_a], add=True)

                @pl.when(c + 2 < s_ch)
                def _():
                    start_loads(c + 2, idx_a, 0)

                wait_loads(idx_b, 1)
                pltpu.sync_copy(m_v.at[1], agg_sh.at[idx_b], add=True)
                return carry

            lax.fori_loop(0, s_ch // 2, body, 0)

        @pl.when(cid == 0)
        def _():
            scatter_from(ma_hbm)

        @pl.when(cid == 1)
        def _():
            scatter_from(mb_hbm)

        plsc.subcore_barrier()

        def writeback(out_hbm):
            def body(kk, carry):
                r = sid * _WB + kk * _SC_CHUNK
                pltpu.sync_copy(agg_sh.at[pl.ds(r, _SC_CHUNK)], m_v.at[0])
                pltpu.sync_copy(m_v.at[0], out_hbm.at[pl.ds(r, _SC_CHUNK)])
                return carry
            lax.fori_loop(0, _WB_CH, body, 0)

        @pl.when(cid == 0)
        def _():
            writeback(aa_hbm)

        @pl.when(cid == 1)
        def _():
            writeback(ab_hbm)

    return k(ma, mb, dstp)


def kernel(node_features, latent_sidechain, t, edge_raw, params, edge_index):
    p = params
    dst = edge_index[0]
    src = edge_index[1]

    pad_e = E_PAD - N_EDGES
    srcp = jnp.concatenate([src, jnp.zeros((pad_e,), jnp.int32)])
    dstp = jnp.concatenate([dst, jnp.full((pad_e,), TRASH_ROW, jnp.int32)])
    edge_raw_p = jnp.concatenate(
        [edge_raw, jnp.zeros((pad_e, 64), jnp.float32)], axis=0)

    Wn1 = p['Wn1']
    Wt, Wnf, Wlat = Wn1[:64], Wn1[64:70], Wn1[70:]
    Wn3 = p['Wn3'].reshape(512, 5, 256).transpose(1, 0, 2)
    bn3 = p['bn3'].reshape(5, 1, 256)
    W_head = p['W_head'].reshape(5, 256, 20)
    W_head = jnp.pad(W_head, ((0, 0), (0, 0), (0, 12)))
    b_head = jnp.pad(p['b_head'], (0, 12)).reshape(1, 32)

    r2 = lambda v: v.reshape(1, -1)
    lws = []
    for lp in p['layers']:
        Wm = lp['Wm']
        lws.append(dict(
            Wms=Wm[:256], Wmd=Wm[256:512], Wme=Wm[512:],
            bm=r2(lp['bm']), WuA=lp['Wu'][:128], WuB=lp['Wu'][128:],
            WeL=lp['We'], gn=r2(lp['gn']), bn=r2(lp['bn']),
            ge=r2(lp['ge']), be=r2(lp['be']),
        ))

    node, ps, pd = _node_embed(
        t.reshape(1, 1), p['w_time'].reshape(1, 32),
        node_features, latent_sidechain,
        Wt, Wnf, Wlat, r2(p['bn1']), p['Wn2'], r2(p['bn2']), Wn3, bn3,
        r2(p['g_node']), r2(p['b_node']),
        lws[0]['Wms'], lws[0]['Wmd'], lws[0]['bm'])

    srcp_h = (srcp[:E_HALF], srcp[E_HALF:])
    dstp_h = (dstp[:E_HALF], dstp[E_HALF:])

    g_h = [_sc_gather(ps, pd, srcp_h[0], dstp_h[0], E_HALF),
           _sc_gather(ps, pd, srcp_h[1], dstp_h[1], E_HALF)]

    e_h = [
        _edge_embed(edge_raw_p[h * E_HALF:(h + 1) * E_HALF],
                    p['We1'], r2(p['be1']), p['We2'],
                    r2(p['be2']), p['We3'], r2(p['be3']),
                    r2(p['g_edge']), r2(p['b_edge']), E_HALF)
        for h in range(2)
    ]

    for l in range(4):
        lw = lws[l]
        agg_h = [None, None]
        for h in range(2):
            ma, mb, e_new = _edge_layer(e_h[h], g_h[h][0], g_h[h][1],
                                        lw['Wme'], lw['WeL'],
                                        lw['ge'], lw['be'], E_HALF)
            e_h[h] = e_new
            agg_h[h] = _sc_scatter(ma, mb, dstp_h[h], E_HALF)
        if l < 3:
            nxt = lws[l + 1]
            node, ps, pd = _node_update(
                node, agg_h[0][0][:N_NODES], agg_h[0][1][:N_NODES],
                agg_h[1][0][:N_NODES], agg_h[1][1][:N_NODES],
                lw['WuA'], lw['WuB'], lw['gn'], lw['bn'],
                nxt['Wms'], nxt['Wmd'], nxt['bm'], final=False)
            g_h = [_sc_gather(ps, pd, srcp_h[0], dstp_h[0], E_HALF),
                   _sc_gather(ps, pd, srcp_h[1], dstp_h[1], E_HALF)]
        else:
            node, logits, _ = _node_update(
                node, agg_h[0][0][:N_NODES], agg_h[0][1][:N_NODES],
                agg_h[1][0][:N_NODES], agg_h[1][1][:N_NODES],
                lw['WuA'], lw['WuB'], lw['gn'], lw['bn'],
                W_head, lws[0]['Wms'], b_head, final=True)

    return logits[:, :20]
